# Initial kernel scaffold; baseline (speedup 1.0000x reference)
import jax
import jax.numpy as jnp
from jax.experimental import pallas as pl


def kernel(x, Wqkv, bqkv, Wproj, bproj):
    raise NotImplementedError("write your pallas kernel here")


# speedup vs baseline: 16.4154x; 16.4154x over previous
import functools

import jax
import jax.numpy as jnp
from jax.experimental import pallas as pl

H = 16


def _qkv_softmax_kernel(x_ref, w_ref, b_ref, t_ref, *, scale):
    xb = x_ref[...]
    dn = (((1,), (1,)), ((), ()))
    q = jax.lax.dot_general(xb, w_ref[0, 0], dn,
                            preferred_element_type=jnp.float32) + b_ref[0, 0, 0]
    k = jax.lax.dot_general(xb, w_ref[1, 0], dn,
                            preferred_element_type=jnp.float32) + b_ref[1, 0, 0]
    v = jax.lax.dot_general(xb, w_ref[2, 0], dn,
                            preferred_element_type=jnp.float32) + b_ref[2, 0, 0]
    m = (q - k) * scale
    m = m - jnp.max(m, axis=-1, keepdims=True)
    e = jnp.exp(m)
    t_ref[...] = (e / jnp.sum(e, axis=-1, keepdims=True)) * v


def _proj_kernel(t_ref, w_ref, b_ref, x_ref, o_ref):
    dn = (((1,), (1,)), ((), ()))
    o_ref[...] = (x_ref[...]
                  + jax.lax.dot_general(t_ref[...], w_ref[...], dn,
                                        preferred_element_type=jnp.float32)
                  + b_ref[0])


def kernel(x, Wqkv, bqkv, Wproj, bproj):
    B, S, D = x.shape
    dh = D // H
    scale = dh ** -0.5
    x2 = x.reshape(S, D)
    w3 = Wqkv.reshape(3, H, dh, D)
    b3 = bqkv.reshape(3, H, 1, dh)

    BS1 = 2048
    t = pl.pallas_call(
        functools.partial(_qkv_softmax_kernel, scale=scale),
        grid=(S // BS1, H),
        in_specs=[
            pl.BlockSpec((BS1, D), lambda i, h: (i, 0)),
            pl.BlockSpec((3, 1, dh, D), lambda i, h: (0, h, 0, 0)),
            pl.BlockSpec((3, 1, 1, dh), lambda i, h: (0, h, 0, 0)),
        ],
        out_specs=pl.BlockSpec((BS1, dh), lambda i, h: (i, h)),
        out_shape=jax.ShapeDtypeStruct((S, D), jnp.float32),
    )(x2, w3, b3)

    BS2 = 512
    out = pl.pallas_call(
        _proj_kernel,
        grid=(S // BS2,),
        in_specs=[
            pl.BlockSpec((BS2, D), lambda i: (i, 0)),
            pl.BlockSpec((D, D), lambda i: (0, 0)),
            pl.BlockSpec((1, D), lambda i: (0, 0)),
            pl.BlockSpec((BS2, D), lambda i: (i, 0)),
        ],
        out_specs=pl.BlockSpec((BS2, D), lambda i: (i, 0)),
        out_shape=jax.ShapeDtypeStruct((S, D), jnp.float32),
    )(t, Wproj, bproj.reshape(1, D), x2)

    return out.reshape(B, S, D)
